# speedup vs baseline: 1.4299x; 1.4299x over previous
import functools

import jax
import jax.numpy as jnp
from jax.experimental import pallas as pl

B, S, D = 1, 2048, 1024
E, FF = 8, 2048
EPS = 0.01
T = B * S
TF = 512
NEG_INF = float("-inf")


def _routing_kernel(x_ref, gw_ref, logits_ref, we_ref):
    x = x_ref[...]
    gw = gw_ref[...]
    s = jax.lax.dot_general(x, gw, (((1,), (1,)), ((), ())),
                            preferred_element_type=jnp.float32)
    logits_ref[...] = s

    iota = jax.lax.broadcasted_iota(jnp.int32, s.shape, 1)

    def softmax(z):
        m = jnp.max(z, axis=-1, keepdims=True)
        ez = jnp.exp(z - m)
        return ez / jnp.sum(ez, axis=-1, keepdims=True)

    def onehot_argmax(z):
        m = jnp.max(z, axis=-1, keepdims=True)
        idx = jnp.min(jnp.where(z == m, iota, E), axis=-1, keepdims=True)
        return iota == idx, m

    oh1, max_val = onehot_argmax(s)
    factor = jnp.maximum(jnp.abs(s), max_val)
    mask1 = ((max_val - s) / factor) > 2.0 * EPS
    gates1 = softmax(jnp.where(mask1, NEG_INF, s))
    m1 = jnp.sum(jnp.where(oh1, gates1, 0.0), axis=-1, keepdims=True)

    masked_scores = jnp.where(oh1, NEG_INF, s)
    oh2, max2 = onehot_argmax(masked_scores)
    factor2 = jnp.maximum(jnp.abs(s), max2)
    mask2 = ((max2 - s) / factor2) > 2.0 * EPS
    gates2 = softmax(jnp.where(mask2, NEG_INF, masked_scores))
    m2 = jnp.sum(jnp.where(oh2, gates2, 0.0), axis=-1, keepdims=True)

    we_ref[...] = jnp.where(oh1, m1, 0.0) + jnp.where(oh2, m2, 0.0)


def _expert_kernel(x_ref, we_ref, w1_ref, w3_ref, w2_ref, out_ref):
    x = x_ref[...]
    g = jax.lax.dot_general(x, w1_ref[0], (((1,), (1,)), ((), ())),
                            preferred_element_type=jnp.float32)
    u = jax.lax.dot_general(x, w3_ref[0], (((1,), (1,)), ((), ())),
                            preferred_element_type=jnp.float32)
    w = we_ref[0, 0, :][:, None]
    h = (g * jax.lax.logistic(g)) * u * w
    acc = jax.lax.dot_general(h, w2_ref[0], (((1,), (1,)), ((), ())),
                              preferred_element_type=jnp.float32)
    first = jnp.logical_and(pl.program_id(0) == 0, pl.program_id(1) == 0)

    @pl.when(first)
    def _():
        out_ref[...] = acc

    @pl.when(jnp.logical_not(first))
    def _():
        out_ref[...] += acc


@functools.partial(jax.jit, static_argnames=())
def kernel(hidden_states, gate_w, w1, w3, w2):
    x = hidden_states.reshape(T, D)
    logits, we = pl.pallas_call(
        _routing_kernel,
        out_shape=(
            jax.ShapeDtypeStruct((T, E), jnp.float32),
            jax.ShapeDtypeStruct((T, E), jnp.float32),
        ),
    )(x, gate_w)

    weT = we.T.reshape(E, 1, T)
    final = pl.pallas_call(
        _expert_kernel,
        grid=(E, FF // TF),
        in_specs=[
            pl.BlockSpec((T, D), lambda e, f: (0, 0)),
            pl.BlockSpec((1, 1, T), lambda e, f: (e, 0, 0)),
            pl.BlockSpec((1, TF, D), lambda e, f: (e, f, 0)),
            pl.BlockSpec((1, TF, D), lambda e, f: (e, f, 0)),
            pl.BlockSpec((1, D, TF), lambda e, f: (e, 0, f)),
        ],
        out_specs=pl.BlockSpec((T, D), lambda e, f: (0, 0)),
        out_shape=jax.ShapeDtypeStruct((T, D), jnp.float32),
    )(x, weT, w1, w3, w2)

    return final.reshape(hidden_states.shape), logits
